# Initial kernel scaffold; baseline (speedup 1.0000x reference)
import jax
import jax.numpy as jnp
from jax.experimental import pallas as pl


def kernel(x, edge_index, edge_attr, params):
    raise NotImplementedError("write your pallas kernel here")


# speedup vs baseline: 3.2840x; 3.2840x over previous
import functools

import jax
import jax.numpy as jnp
from jax import lax
from jax.experimental import pallas as pl
from jax.experimental.pallas import tpu as pltpu
from jax.experimental.pallas import tpu_sc as plsc

N = 10000
E = 320000
D_IN = 128
E_IN = 16
HID = 128
NLAYERS = 4

NC = 2
NS = 16
NW = NC * NS
K = 128
NCHUNK = E // K
CPW = -(-NCHUNK // NW)
RPS = N // NS
EAW = 32
SW = NLAYERS * EAW

_sc_mesh = plsc.VectorSubcoreMesh(
    core_axis_name="c", subcore_axis_name="s", num_cores=NC, num_subcores=NS)


def _zero_rows(rows_v, nrow, width):
  z = jnp.zeros((16,), jnp.float32)

  def body(i, _):
    for j in range(width // 16):
      rows_v[i, pl.ds(j * 16, 16)] = z
    return 0

  lax.fori_loop(0, nrow, body, 0, unroll=2)


@functools.partial(
    pl.kernel,
    out_type=jax.ShapeDtypeStruct((NC, NS, RPS, SW), jnp.float32),
    mesh=_sc_mesh,
    scratch_types=[
        pltpu.VMEM((K,), jnp.int32),
        pltpu.VMEM((NLAYERS, K), jnp.float32),
        pltpu.VMEM((K, EAW), jnp.float32),
        pltpu.VMEM((K, SW), jnp.float32),
        pltpu.VMEM_SHARED((N, SW), jnp.float32),
    ],
)
def _sc_phase_a(ea_hbm, dst_hbm, y_hbm, out_hbm, dst_v, y_v, ea_v, val_v,
                acc_sh):
  cid = lax.axis_index("c")
  sid = lax.axis_index("s")
  wid = sid * NC + cid

  if True:
    _zero_rows(val_v, K, SW)
    for t in range(5):
      pltpu.sync_copy(val_v.at[pl.ds(0, RPS // 5)],
                      acc_sh.at[pl.ds(sid * RPS + t * (RPS // 5), RPS // 5)])
    plsc.subcore_barrier()

    def chunk(j, _):
      c = wid + j * NW

      @pl.when(c < NCHUNK)
      def _():
        base = c * K
        pltpu.sync_copy(dst_hbm.at[pl.ds(base, K)], dst_v)
        for l in range(NLAYERS):
          pltpu.sync_copy(y_hbm.at[l, pl.ds(base, K)], y_v.at[l])
        pltpu.sync_copy(ea_hbm.at[pl.ds(base, K)], ea_v)

        def edge16(i, _):
          yv = [y_v[l, pl.ds(i * 16, 16)] for l in range(NLAYERS)]
          for e in range(16):
            row = i * 16 + e
            a0 = ea_v[row, pl.ds(0, 16)]
            a1 = ea_v[row, pl.ds(16, 16)]
            for l in range(NLAYERS):
              yl = yv[l][e]
              val_v[row, pl.ds(l * EAW, 16)] = a0 * yl
              val_v[row, pl.ds(l * EAW + 16, 16)] = a1 * yl
          return 0

        lax.fori_loop(0, K // 16, edge16, 0)
        pltpu.sync_copy(val_v, acc_sh.at[dst_v], add=True)

      return 0

    lax.fori_loop(0, CPW, chunk, 0)
    plsc.subcore_barrier()
    pltpu.sync_copy(acc_sh.at[pl.ds(sid * RPS, RPS)], out_hbm.at[cid, sid])


@functools.partial(
    pl.kernel,
    out_type=jax.ShapeDtypeStruct((NC, NS, RPS, HID), jnp.float32),
    mesh=_sc_mesh,
    scratch_types=[
        pltpu.VMEM((K,), jnp.int32),
        pltpu.VMEM((K,), jnp.int32),
        pltpu.VMEM((K,), jnp.float32),
        pltpu.VMEM((K, HID), jnp.float32),
        pltpu.VMEM_SHARED((N, HID), jnp.float32),
        pltpu.SemaphoreType.DMA,
    ],
)
def _sc_spmm(h_hbm, src_hbm, dst_hbm, y_hbm, out_hbm,
             src_v, dst_v, y_v, rows_v, acc_sh, sem):
  cid = lax.axis_index("c")
  sid = lax.axis_index("s")
  wid = sid * NC + cid

  if True:
    _zero_rows(rows_v, K, HID)
    for t in range(5):
      pltpu.sync_copy(rows_v.at[pl.ds(0, RPS // 5)],
                      acc_sh.at[pl.ds(sid * RPS + t * (RPS // 5), RPS // 5)])
    plsc.subcore_barrier()

    def chunk(j, _):
      c = wid + j * NW

      @pl.when(c < NCHUNK)
      def _():
        base = c * K
        pltpu.sync_copy(src_hbm.at[pl.ds(base, K)], src_v)
        pltpu.sync_copy(dst_hbm.at[pl.ds(base, K)], dst_v)
        pltpu.sync_copy(y_hbm.at[pl.ds(base, K)], y_v)
        pltpu.async_copy(h_hbm.at[src_v], rows_v, sem).wait()

        def edge16(i, _):
          yv = y_v[pl.ds(i * 16, 16)]
          for e in range(16):
            row = i * 16 + e
            ye = yv[e]
            for jj in range(HID // 16):
              sl = pl.ds(jj * 16, 16)
              rows_v[row, sl] = rows_v[row, sl] * ye
          return 0

        lax.fori_loop(0, K // 16, edge16, 0)
        pltpu.sync_copy(rows_v, acc_sh.at[dst_v], add=True)

      return 0

    lax.fori_loop(0, CPW, chunk, 0)
    plsc.subcore_barrier()
    pltpu.sync_copy(acc_sh.at[pl.ds(sid * RPS, RPS)], out_hbm.at[cid, sid])


_EB = 2560
_NB = 1000


def _tc_y_body(eattr_ref, wemb_ref, bemb_ref, a4_ref, d_ref, y_ref, ea2_ref):
  eattr = eattr_ref[...]
  ea = jnp.dot(eattr, wemb_ref[...],
               preferred_element_type=jnp.float32) + bemb_ref[...]
  z = lax.dot_general(a4_ref[...], ea, (((1,), (1,)), ((), ())),
                      preferred_element_type=jnp.float32)
  y_ref[...] = jax.nn.sigmoid(z + d_ref[...])
  ones = jnp.ones((_EB, 1), jnp.float32)
  zeros = jnp.zeros((_EB, EAW - E_IN - 1), jnp.float32)
  ea2_ref[...] = jnp.concatenate([eattr, ones, zeros], axis=1)


def _tc_y(edge_attr, Wemb, bemb, A4T, dvec):
  return pl.pallas_call(
      _tc_y_body,
      grid=(E // _EB,),
      in_specs=[
          pl.BlockSpec((_EB, E_IN), lambda i: (i, 0)),
          pl.BlockSpec((E_IN, HID), lambda i: (0, 0)),
          pl.BlockSpec((1, HID), lambda i: (0, 0)),
          pl.BlockSpec((NLAYERS, HID), lambda i: (0, 0)),
          pl.BlockSpec((NLAYERS, 1), lambda i: (0, 0)),
      ],
      out_specs=[
          pl.BlockSpec((NLAYERS, _EB), lambda i: (0, i)),
          pl.BlockSpec((_EB, EAW), lambda i: (i, 0)),
      ],
      out_shape=[
          jax.ShapeDtypeStruct((NLAYERS, E), jnp.float32),
          jax.ShapeDtypeStruct((E, EAW), jnp.float32),
      ],
  )(edge_attr, Wemb, bemb.reshape(1, HID), A4T, dvec.reshape(NLAYERS, 1))


def _tc_embed_body(x_ref, w_ref, b_ref, h_ref):
  h_ref[...] = jnp.dot(x_ref[...], w_ref[...],
                       preferred_element_type=jnp.float32) + b_ref[...]


def _tc_embed(x, W, b):
  return pl.pallas_call(
      _tc_embed_body,
      grid=(N // _NB,),
      in_specs=[
          pl.BlockSpec((_NB, D_IN), lambda i: (i, 0)),
          pl.BlockSpec((D_IN, HID), lambda i: (0, 0)),
          pl.BlockSpec((1, HID), lambda i: (0, 0)),
      ],
      out_specs=pl.BlockSpec((_NB, HID), lambda i: (i, 0)),
      out_shape=jax.ShapeDtypeStruct((N, HID), jnp.float32),
  )(x, W, b.reshape(1, HID))


def _tc_layer_body(l, gp_ref, s_ref, h_ref, ml_ref, beff_ref,
                   g_ref, ho_ref):
  g = gp_ref[0] + gp_ref[1]
  sboth = s_ref[0] + s_ref[1]
  s_l = sboth[:, l * EAW:l * EAW + E_IN]
  deg = sboth[:, l * EAW + E_IN:l * EAW + E_IN + 1]
  agg = (g + jnp.dot(s_l, ml_ref[...], preferred_element_type=jnp.float32)
         + deg * beff_ref[...])
  mu = jnp.mean(agg, axis=-1, keepdims=True)
  var = jnp.mean((agg - mu) ** 2, axis=-1, keepdims=True)
  ln = ((agg - mu) / jnp.sqrt(var + 1e-5)) * g_ref[...][0:1] + g_ref[...][1:2]
  ho_ref[...] = h_ref[...] + jnp.maximum(ln, 0.0)


def _tc_layer(l, Gp, S2, h, Ml, beff, ln_gb):
  return pl.pallas_call(
      functools.partial(_tc_layer_body, l),
      grid=(N // _NB,),
      in_specs=[
          pl.BlockSpec((NC, _NB, HID), lambda i: (0, i, 0)),
          pl.BlockSpec((NC, _NB, SW), lambda i: (0, i, 0)),
          pl.BlockSpec((_NB, HID), lambda i: (i, 0)),
          pl.BlockSpec((E_IN, HID), lambda i: (0, 0)),
          pl.BlockSpec((1, HID), lambda i: (0, 0)),
          pl.BlockSpec((2, HID), lambda i: (0, 0)),
      ],
      out_specs=pl.BlockSpec((_NB, HID), lambda i: (i, 0)),
      out_shape=jax.ShapeDtypeStruct((N, HID), jnp.float32),
  )(Gp, S2, h, Ml, beff.reshape(1, HID), ln_gb)


def _tc_head_body(h_ref, w1_ref, b1_ref, w2_ref, b2_ref, w3_ref, b3_ref,
                  o_ref):
  m = jnp.maximum(jnp.dot(h_ref[...], w1_ref[...],
                          preferred_element_type=jnp.float32) + b1_ref[...], 0.)
  m = jnp.maximum(jnp.dot(m, w2_ref[...],
                          preferred_element_type=jnp.float32) + b2_ref[...], 0.)
  o = jnp.dot(m, w3_ref[...], preferred_element_type=jnp.float32) + b3_ref[...]
  v = o[:, 0:1]
  s = o[:, 1:2]
  c = o[:, 2:3]
  norm = jnp.sqrt(s * s + c * c + 1e-8)
  o_ref[...] = jnp.concatenate(
      [v, s / norm, c / norm, o[:, 3:]], axis=1)


def _tc_head(h, W1, b1, W2, b2, W3, b3):
  return pl.pallas_call(
      _tc_head_body,
      grid=(N // _NB,),
      in_specs=[
          pl.BlockSpec((_NB, HID), lambda i: (i, 0)),
          pl.BlockSpec((HID, HID), lambda i: (0, 0)),
          pl.BlockSpec((1, HID), lambda i: (0, 0)),
          pl.BlockSpec((HID, HID // 2), lambda i: (0, 0)),
          pl.BlockSpec((1, HID // 2), lambda i: (0, 0)),
          pl.BlockSpec((HID // 2, 8), lambda i: (0, 0)),
          pl.BlockSpec((1, 8), lambda i: (0, 0)),
      ],
      out_specs=pl.BlockSpec((_NB, 8), lambda i: (i, 0)),
      out_shape=jax.ShapeDtypeStruct((N, 8), jnp.float32),
  )(h, W1, b1.reshape(1, HID), W2, b2.reshape(1, HID // 2), W3, b3)


def kernel(x, edge_index, edge_attr, params):
  p = params
  src = edge_index[0]
  dst = edge_index[1]
  Wemb, bemb = p['edge_embed']
  hi = functools.partial(jnp.dot, precision=jax.lax.Precision.HIGHEST)

  A4T = jnp.concatenate([lp['adm'][0] for lp in p['layers']], axis=1).T
  dvec = jnp.stack([lp['adm'][1][0] for lp in p['layers']])
  Mls = [hi(Wemb, lp['lin_edge'][0]) for lp in p['layers']]
  beffs = [hi(bemb, lp['lin_edge'][0]) + lp['lin_edge'][1]
           for lp in p['layers']]
  ln_gbs = [jnp.stack([lp['ln'][0], lp['ln'][1]]) for lp in p['layers']]

  Y, ea2 = _tc_y(edge_attr, Wemb, bemb, A4T, dvec)
  S2 = _sc_phase_a(ea2, dst, Y).reshape(NC, N, SW)
  h = _tc_embed(x, p['node_embed'][0], p['node_embed'][1])

  for l, lp in enumerate(p['layers']):
    hn = _tc_embed(h, lp['lin_node'][0], lp['lin_node'][1])
    Gp = _sc_spmm(hn, src, dst, Y[l]).reshape(NC, N, HID)
    h = _tc_layer(l, Gp, S2, h, Mls[l], beffs[l], ln_gbs[l])

  hp = p['head']
  W3 = jnp.concatenate([hp['v'][0], hp['s'][0], hp['c'][0],
                        jnp.zeros((HID // 2, 5), jnp.float32)], axis=1)
  b3 = jnp.concatenate([hp['v'][1], hp['s'][1], hp['c'][1],
                        jnp.zeros((5,), jnp.float32)]).reshape(1, 8)
  o = _tc_head(h, hp['mlp1'][0], hp['mlp1'][1], hp['mlp2'][0], hp['mlp2'][1],
               W3, b3)
  return (o[:, 0], o[:, 1], o[:, 2])
